# baseline (speedup 1.0000x reference)
import functools

import jax
import jax.numpy as jnp
from jax import lax
from jax.experimental import pallas as pl
from jax.experimental.pallas import tpu as pltpu
from jax.experimental.pallas import tpu_sc as plsc

N = 524288
C = 19
B = 64
HB = C * B
HB2 = 2 * HB
LHB2 = HB2 * 32
M = N // 2
BKL = 4096
NW = 32
CH = 512
SS = (M // NW) // CH
FGSLICE = M // NW


def _bucket_half(x, lab):
    ex = jnp.exp(x)
    p = ex / jnp.sum(ex, axis=0, keepdims=True)
    ci = lax.broadcasted_iota(jnp.int32, (C, BKL), 0)
    oh = ci == lab[None, :]
    e = jnp.where(oh, 1.0 - p, p)
    b = jnp.minimum((e * B).astype(jnp.int32), B - 1)
    idx = ci * B + b
    idxfg = jnp.sum(jnp.where(oh, idx, 0), axis=0)
    return idx, idxfg


def tc1_body(lo_ref, hi_ref, lab_lo_ref, lab_hi_ref, pk_ref, pkfg_ref):
    idx_lo, fg_lo = _bucket_half(lo_ref[...], lab_lo_ref[...].reshape(BKL))
    idx_hi, fg_hi = _bucket_half(hi_ref[...], lab_hi_ref[...].reshape(BKL))
    pk_ref[...] = idx_lo | (idx_hi << 16)
    pkfg_ref[...] = (fg_lo | (fg_hi << 16)).reshape(1, 1, BKL)


def sc_body(pk_hbm, pkfg_hbm, out_hbm, st0, st1, fgbuf, hist, sem0, sem1):
    wid = lax.axis_index("s") * 2 + lax.axis_index("c")

    zero16 = jnp.zeros((16,), jnp.float32)

    @functools.partial(plsc.parallel_loop, 0, LHB2 // 16, unroll=8)
    def _zero(i):
        hist[pl.ds(i * 16, 16)] = zero16

    one16 = jnp.ones((16,), jnp.float32)
    lane = lax.broadcasted_iota(jnp.int32, (16,), 0)
    lane2 = lane * 2

    def scat_stage(stage):
        @functools.partial(plsc.parallel_loop, 0, (C * CH) // 16, unroll=8)
        def _body(i):
            r = i >> 5
            col = (i & 31) * 16
            w = stage[r, pl.ds(col, 16)]
            lo = jnp.bitwise_and(w, 0xFFFF)
            hi = lax.shift_right_logical(w, 16)
            plsc.addupdate_scatter(hist, [(lo << 5) | lane2], one16)
            plsc.addupdate_scatter(hist, [(hi << 5) | lane2 | 1], one16)

    sems = [sem0, sem1]
    stages = [st0, st1]

    def start(slot, s):
        col = wid * (SS * CH) + s * CH
        return pltpu.async_copy(
            pk_hbm.at[:, pl.ds(col, CH)], stages[slot], sems[slot])

    for s in range(SS):
        col = wid * (SS * CH) + s * CH
        pltpu.sync_copy(pk_hbm.at[:, pl.ds(col, CH)], st0)
        scat_stage(st0)

    pltpu.sync_copy(pkfg_hbm.at[pl.ds(wid * FGSLICE, FGSLICE)], fgbuf)

    @functools.partial(plsc.parallel_loop, 0, FGSLICE // 16, unroll=8)
    def _fg(i):
        w = fgbuf[pl.ds(i * 16, 16)]
        lo = jnp.bitwise_and(w, 0xFFFF) + HB
        hi = lax.shift_right_logical(w, 16) + HB
        plsc.addupdate_scatter(hist, [(lo << 5) | lane2], one16)
        plsc.addupdate_scatter(hist, [(hi << 5) | lane2 | 1], one16)

    pltpu.sync_copy(hist, out_hbm.at[wid])


def fin_body(h_ref, o_ref):
    h = h_ref[...]
    t = jnp.sum(h, axis=0)
    gi = lax.broadcasted_iota(jnp.int32, (B * 32, B), 0)
    gj = lax.broadcasted_iota(jnp.int32, (B * 32, B), 1)
    gs = ((gi >> 5) == gj).astype(jnp.float32)
    cnt = lax.dot_general(t[0], gs, (((1,), (0,)), ((), ())),
                          preferred_element_type=jnp.float32,
                          precision=lax.Precision.HIGHEST)
    nfg = lax.dot_general(t[1], gs, (((1,), (0,)), ((), ())),
                          preferred_element_type=jnp.float32,
                          precision=lax.Precision.HIGHEST)
    bi = lax.broadcasted_iota(jnp.int32, (C, B), 1).astype(jnp.float32)
    v = (bi + 0.5) * (1.0 / B)
    g = jnp.sum(nfg, axis=1, keepdims=True)
    ii = lax.broadcasted_iota(jnp.int32, (B, B), 0)
    jj = lax.broadcasted_iota(jnp.int32, (B, B), 1)
    tri = (ii <= jj).astype(jnp.float32)
    cum = lax.dot_general(nfg, tri, (((1,), (0,)), ((), ())),
                          preferred_element_type=jnp.float32,
                          precision=lax.Precision.HIGHEST)
    cfg = g - cum
    s = jnp.sum(v * cnt, axis=1, keepdims=True)
    sfg = jnp.sum(v * nfg, axis=1, keepdims=True)
    w = (sfg + jnp.sum(v * cnt * cfg, axis=1, keepdims=True)
         + jnp.sum(v * nfg * (cnt - 1.0), axis=1, keepdims=True) * 0.5)
    loss = s - w / jnp.maximum(g, 1.0)
    present = (g > 0.0).astype(jnp.float32)
    num = jnp.sum(loss * present)
    den = jnp.maximum(jnp.sum(present), 1.0)
    o_ref[...] = (num / den).reshape(1, 1)


_GRID1 = M // BKL

_tc1 = pl.pallas_call(
    tc1_body,
    grid=(_GRID1,),
    in_specs=[
        pl.BlockSpec((C, BKL), lambda i: (0, i)),
        pl.BlockSpec((C, BKL), lambda i: (0, i + _GRID1)),
        pl.BlockSpec((1, 1, BKL), lambda i: (i, 0, 0)),
        pl.BlockSpec((1, 1, BKL), lambda i: (i + _GRID1, 0, 0)),
    ],
    out_specs=[
        pl.BlockSpec((C, BKL), lambda i: (0, i)),
        pl.BlockSpec((1, 1, BKL), lambda i: (i, 0, 0)),
    ],
    out_shape=[
        jax.ShapeDtypeStruct((C, M), jnp.int32),
        jax.ShapeDtypeStruct((_GRID1, 1, BKL), jnp.int32),
    ],
)

@functools.cache
def _sc_hist():
    return pl.kernel(
        sc_body,
        out_type=jax.ShapeDtypeStruct((NW, LHB2), jnp.float32),
        mesh=plsc.VectorSubcoreMesh(core_axis_name="c", subcore_axis_name="s"),
        compiler_params=pltpu.CompilerParams(needs_layout_passes=False),
        scratch_types=[
            pltpu.VMEM((C, CH), jnp.int32),
            pltpu.VMEM((C, CH), jnp.int32),
            pltpu.VMEM((FGSLICE,), jnp.int32),
            pltpu.VMEM((LHB2,), jnp.float32),
            pltpu.SemaphoreType.DMA,
            pltpu.SemaphoreType.DMA,
        ],
    )

_fin = pl.pallas_call(
    fin_body,
    out_shape=jax.ShapeDtypeStruct((1, 1), jnp.float32),
)


def kernel(logits, labels):
    lt = logits.T
    labels3 = labels.reshape(N // BKL, 1, BKL)
    pk, pkfg3 = _tc1(lt, lt, labels3, labels3)
    hist = _sc_hist()(pk, pkfg3.reshape(M))
    out = _fin(hist.reshape(NW, 2, C, B * 32))
    return out.reshape(())


# speedup vs baseline: 1.1340x; 1.1340x over previous
import functools

import jax
import jax.numpy as jnp
from jax import lax
from jax.experimental import pallas as pl
from jax.experimental.pallas import tpu as pltpu
from jax.experimental.pallas import tpu_sc as plsc

N = 524288
C = 19
B = 64
HB = C * B
HB2 = 2 * HB
LHB2 = HB2 * 32
M = N // 2
BKL = 8192
NW = 32
CH = 512
SS = (M // NW) // CH
FGSLICE = M // NW


def _bucket_half(x, lab):
    ex = jnp.exp(x)
    rs = float(B) / jnp.sum(ex, axis=0, keepdims=True)
    bp = jnp.minimum((ex * rs).astype(jnp.int32), B - 1)
    ci = lax.broadcasted_iota(jnp.int32, (C, BKL), 0)
    oh = ci == lab[None, :]
    b = jnp.where(oh, (B - 1) - bp, bp)
    idx = ci * B + b
    idxfg = jnp.sum(jnp.where(oh, idx, 0), axis=0)
    return idx, idxfg


def tc1_body(lo_ref, hi_ref, lab_lo_ref, lab_hi_ref, pk_ref, pkfg_ref):
    idx_lo, fg_lo = _bucket_half(lo_ref[...], lab_lo_ref[...].reshape(BKL))
    idx_hi, fg_hi = _bucket_half(hi_ref[...], lab_hi_ref[...].reshape(BKL))
    pk_ref[...] = idx_lo | (idx_hi << 16)
    pkfg_ref[...] = (fg_lo | (fg_hi << 16)).reshape(1, 1, BKL)


def sc_body(pk_hbm, pkfg_hbm, out_hbm, st0, st1, fgbuf, hist, sem0, sem1):
    wid = lax.axis_index("s") * 2 + lax.axis_index("c")

    zero16 = jnp.zeros((16,), jnp.float32)

    @functools.partial(plsc.parallel_loop, 0, LHB2 // 16, unroll=8)
    def _zero(i):
        hist[pl.ds(i * 16, 16)] = zero16

    one16 = jnp.ones((16,), jnp.float32)
    lane = lax.broadcasted_iota(jnp.int32, (16,), 0)
    lane2 = lane * 2

    def scat_stage(stage):
        @functools.partial(plsc.parallel_loop, 0, (C * CH) // 16, unroll=8)
        def _body(i):
            r = i >> 5
            col = (i & 31) * 16
            w = stage[r, pl.ds(col, 16)]
            lo = jnp.bitwise_and(w, 0xFFFF)
            hi = lax.shift_right_logical(w, 16)
            plsc.addupdate_scatter(hist, [(lo << 5) | lane2], one16)
            plsc.addupdate_scatter(hist, [(hi << 5) | lane2 | 1], one16)

    sems = [sem0, sem1]
    stages = [st0, st1]

    def start(slot, s):
        col = wid * (SS * CH) + s * CH
        return pltpu.async_copy(
            pk_hbm.at[:, pl.ds(col, CH)], stages[slot], sems[slot])

    for s in range(SS):
        col = wid * (SS * CH) + s * CH
        pltpu.sync_copy(pk_hbm.at[:, pl.ds(col, CH)], st0)
        scat_stage(st0)

    pltpu.sync_copy(pkfg_hbm.at[pl.ds(wid * FGSLICE, FGSLICE)], fgbuf)

    @functools.partial(plsc.parallel_loop, 0, FGSLICE // 16, unroll=8)
    def _fg(i):
        w = fgbuf[pl.ds(i * 16, 16)]
        lo = jnp.bitwise_and(w, 0xFFFF) + HB
        hi = lax.shift_right_logical(w, 16) + HB
        plsc.addupdate_scatter(hist, [(lo << 5) | lane2], one16)
        plsc.addupdate_scatter(hist, [(hi << 5) | lane2 | 1], one16)

    pltpu.sync_copy(hist, out_hbm.at[wid])


def fin_body(h_ref, o_ref):
    h = h_ref[...]
    t = jnp.sum(h, axis=0)
    gi = lax.broadcasted_iota(jnp.int32, (B * 32, B), 0)
    gj = lax.broadcasted_iota(jnp.int32, (B * 32, B), 1)
    gs = ((gi >> 5) == gj).astype(jnp.float32)
    cnt = lax.dot_general(t[0], gs, (((1,), (0,)), ((), ())),
                          preferred_element_type=jnp.float32,
                          precision=lax.Precision.HIGHEST)
    nfg = lax.dot_general(t[1], gs, (((1,), (0,)), ((), ())),
                          preferred_element_type=jnp.float32,
                          precision=lax.Precision.HIGHEST)
    bi = lax.broadcasted_iota(jnp.int32, (C, B), 1).astype(jnp.float32)
    v = (bi + 0.5) * (1.0 / B)
    g = jnp.sum(nfg, axis=1, keepdims=True)
    ii = lax.broadcasted_iota(jnp.int32, (B, B), 0)
    jj = lax.broadcasted_iota(jnp.int32, (B, B), 1)
    tri = (ii <= jj).astype(jnp.float32)
    cum = lax.dot_general(nfg, tri, (((1,), (0,)), ((), ())),
                          preferred_element_type=jnp.float32,
                          precision=lax.Precision.HIGHEST)
    cfg = g - cum
    s = jnp.sum(v * cnt, axis=1, keepdims=True)
    sfg = jnp.sum(v * nfg, axis=1, keepdims=True)
    w = (sfg + jnp.sum(v * cnt * cfg, axis=1, keepdims=True)
         + jnp.sum(v * nfg * (cnt - 1.0), axis=1, keepdims=True) * 0.5)
    loss = s - w / jnp.maximum(g, 1.0)
    present = (g > 0.0).astype(jnp.float32)
    num = jnp.sum(loss * present)
    den = jnp.maximum(jnp.sum(present), 1.0)
    o_ref[...] = (num / den).reshape(1, 1)


_GRID1 = M // BKL

_tc1 = pl.pallas_call(
    tc1_body,
    grid=(_GRID1,),
    in_specs=[
        pl.BlockSpec((C, BKL), lambda i: (0, i)),
        pl.BlockSpec((C, BKL), lambda i: (0, i + _GRID1)),
        pl.BlockSpec((1, 1, BKL), lambda i: (i, 0, 0)),
        pl.BlockSpec((1, 1, BKL), lambda i: (i + _GRID1, 0, 0)),
    ],
    out_specs=[
        pl.BlockSpec((C, BKL), lambda i: (0, i)),
        pl.BlockSpec((1, 1, BKL), lambda i: (i, 0, 0)),
    ],
    out_shape=[
        jax.ShapeDtypeStruct((C, M), jnp.int32),
        jax.ShapeDtypeStruct((_GRID1, 1, BKL), jnp.int32),
    ],
)

@functools.cache
def _sc_hist():
    return pl.kernel(
        sc_body,
        out_type=jax.ShapeDtypeStruct((NW, LHB2), jnp.float32),
        mesh=plsc.VectorSubcoreMesh(core_axis_name="c", subcore_axis_name="s"),
        compiler_params=pltpu.CompilerParams(needs_layout_passes=False),
        scratch_types=[
            pltpu.VMEM((C, CH), jnp.int32),
            pltpu.VMEM((C, CH), jnp.int32),
            pltpu.VMEM((FGSLICE,), jnp.int32),
            pltpu.VMEM((LHB2,), jnp.float32),
            pltpu.SemaphoreType.DMA,
            pltpu.SemaphoreType.DMA,
        ],
    )

_fin = pl.pallas_call(
    fin_body,
    out_shape=jax.ShapeDtypeStruct((1, 1), jnp.float32),
)


def kernel(logits, labels):
    lt = logits.T
    labels3 = labels.reshape(N // BKL, 1, BKL)
    pk, pkfg3 = _tc1(lt, lt, labels3, labels3)
    hist = _sc_hist()(pk, pkfg3.reshape(M))
    out = _fin(hist.reshape(NW, 2, C, B * 32))
    return out.reshape(())


# speedup vs baseline: 1.2689x; 1.1190x over previous
import functools

import jax
import jax.numpy as jnp
from jax import lax
from jax.experimental import pallas as pl
from jax.experimental.pallas import tpu as pltpu
from jax.experimental.pallas import tpu_sc as plsc

N = 524288
C = 19
B = 64
HB = C * B
HB2 = 2 * HB
LHB2 = HB2 * 32
M = N // 2
BKL = 8192
NW = 32
CH = 512
SS = (M // NW) // CH
FGSLICE = M // NW


def _bucket_half(x, lab):
    ex = jnp.exp(x)
    rs = float(B) / jnp.sum(ex, axis=0, keepdims=True)
    bp = jnp.minimum((ex * rs).astype(jnp.int32), B - 1)
    ci = lax.broadcasted_iota(jnp.int32, (C, BKL), 0)
    oh = ci == lab[None, :]
    b = jnp.where(oh, (B - 1) - bp, bp)
    idx = ci * B + b
    idxfg = jnp.sum(jnp.where(oh, idx, 0), axis=0)
    return idx, idxfg


def tc1_body(lo_ref, hi_ref, lab_lo_ref, lab_hi_ref, pk_ref, pkfg_ref):
    idx_lo, fg_lo = _bucket_half(lo_ref[...], lab_lo_ref[...].reshape(BKL))
    idx_hi, fg_hi = _bucket_half(hi_ref[...], lab_hi_ref[...].reshape(BKL))
    pk_ref[...] = idx_lo | (idx_hi << 16)
    pkfg_ref[...] = (fg_lo | (fg_hi << 16)).reshape(1, 1, BKL)


def sc_body(pk_hbm, pkfg_hbm, out_hbm, st0, st1, fgbuf, hist, sem0, sem1):
    wid = lax.axis_index("s") * 2 + lax.axis_index("c")

    zero16 = jnp.zeros((16,), jnp.float32)

    @functools.partial(plsc.parallel_loop, 0, LHB2 // 16, unroll=8)
    def _zero(i):
        hist[i >> 7, pl.ds((i & 127) * 16, 16)] = zero16

    one16 = jnp.ones((16,), jnp.float32)
    lane = lax.broadcasted_iota(jnp.int32, (16,), 0)
    lane2 = lane * 2

    def scat_stage(stage):
        @functools.partial(plsc.parallel_loop, 0, (C * CH) // 16, unroll=8)
        def _body(i):
            r = i >> 5
            col = (i & 31) * 16
            w = stage[r, pl.ds(col, 16)]
            lo = jnp.bitwise_and(w, 0xFFFF)
            hi = lax.shift_right_logical(w, 16)
            sl = (lo << 5) | lane2
            sh = (hi << 5) | lane2 | 1
            plsc.addupdate_scatter(hist, [sl >> 11, sl & 2047], one16)
            plsc.addupdate_scatter(hist, [sh >> 11, sh & 2047], one16)

    sems = [sem0, sem1]
    stages = [st0, st1]

    def start(slot, s):
        col = wid * (SS * CH) + s * CH
        return pltpu.async_copy(
            pk_hbm.at[:, pl.ds(col, CH)], stages[slot], sems[slot])

    for s in range(SS):
        col = wid * (SS * CH) + s * CH
        pltpu.sync_copy(pk_hbm.at[:, pl.ds(col, CH)], st0)
        scat_stage(st0)

    pltpu.sync_copy(pkfg_hbm.at[pl.ds(wid * FGSLICE, FGSLICE)], fgbuf)

    @functools.partial(plsc.parallel_loop, 0, FGSLICE // 16, unroll=8)
    def _fg(i):
        w = fgbuf[pl.ds(i * 16, 16)]
        lo = jnp.bitwise_and(w, 0xFFFF) + HB
        hi = lax.shift_right_logical(w, 16) + HB
        sl = (lo << 5) | lane2
        sh = (hi << 5) | lane2 | 1
        plsc.addupdate_scatter(hist, [sl >> 11, sl & 2047], one16)
        plsc.addupdate_scatter(hist, [sh >> 11, sh & 2047], one16)

    pltpu.sync_copy(hist, out_hbm.at[wid])


def _dot(a, b):
    return lax.dot_general(a, b, (((1,), (0,)), ((), ())),
                           preferred_element_type=jnp.float32,
                           precision=lax.Precision.HIGHEST)


def fin_body(h_ref, o_ref):
    h = h_ref[...]
    t = jnp.sum(h, axis=0)
    si = lax.broadcasted_iota(jnp.int32, (C, 2 * C), 0)
    sj = lax.broadcasted_iota(jnp.int32, (C, 2 * C), 1)
    sel0 = (si == sj).astype(jnp.float32)
    sel1 = ((si + C) == sj).astype(jnp.float32)
    gi = lax.broadcasted_iota(jnp.int32, (B * 32, B), 0)
    gj = lax.broadcasted_iota(jnp.int32, (B * 32, B), 1)
    gs = ((gi >> 5) == gj).astype(jnp.float32)
    cnt = _dot(_dot(sel0, t), gs)
    nfg = _dot(_dot(sel1, t), gs)
    bi = lax.broadcasted_iota(jnp.int32, (C, B), 1).astype(jnp.float32)
    v = (bi + 0.5) * (1.0 / B)
    g = jnp.sum(nfg, axis=1, keepdims=True)
    ii = lax.broadcasted_iota(jnp.int32, (B, B), 0)
    jj = lax.broadcasted_iota(jnp.int32, (B, B), 1)
    tri = (ii <= jj).astype(jnp.float32)
    cfg = g - _dot(nfg, tri)
    s = jnp.sum(v * cnt, axis=1, keepdims=True)
    sfg = jnp.sum(v * nfg, axis=1, keepdims=True)
    w = (sfg + jnp.sum(v * cnt * cfg, axis=1, keepdims=True)
         + jnp.sum(v * nfg * (cnt - 1.0), axis=1, keepdims=True) * 0.5)
    loss = s - w / jnp.maximum(g, 1.0)
    present = (g > 0.0).astype(jnp.float32)
    num = jnp.sum(loss * present)
    den = jnp.maximum(jnp.sum(present), 1.0)
    o_ref[...] = (num / den).reshape(1, 1)


_GRID1 = M // BKL

_tc1 = pl.pallas_call(
    tc1_body,
    grid=(_GRID1,),
    in_specs=[
        pl.BlockSpec((C, BKL), lambda i: (0, i)),
        pl.BlockSpec((C, BKL), lambda i: (0, i + _GRID1)),
        pl.BlockSpec((1, 1, BKL), lambda i: (i, 0, 0)),
        pl.BlockSpec((1, 1, BKL), lambda i: (i + _GRID1, 0, 0)),
    ],
    out_specs=[
        pl.BlockSpec((C, BKL), lambda i: (0, i)),
        pl.BlockSpec((1, 1, BKL), lambda i: (i, 0, 0)),
    ],
    out_shape=[
        jax.ShapeDtypeStruct((C, M), jnp.int32),
        jax.ShapeDtypeStruct((_GRID1, 1, BKL), jnp.int32),
    ],
)

@functools.cache
def _sc_hist():
    return pl.kernel(
        sc_body,
        out_type=jax.ShapeDtypeStruct((NW, 2 * C, B * 32), jnp.float32),
        mesh=plsc.VectorSubcoreMesh(core_axis_name="c", subcore_axis_name="s"),
        compiler_params=pltpu.CompilerParams(needs_layout_passes=False),
        scratch_types=[
            pltpu.VMEM((C, CH), jnp.int32),
            pltpu.VMEM((C, CH), jnp.int32),
            pltpu.VMEM((FGSLICE,), jnp.int32),
            pltpu.VMEM((2 * C, B * 32), jnp.float32),
            pltpu.SemaphoreType.DMA,
            pltpu.SemaphoreType.DMA,
        ],
    )

_fin = pl.pallas_call(
    fin_body,
    out_shape=jax.ShapeDtypeStruct((1, 1), jnp.float32),
)


def kernel(logits, labels):
    lt = logits.T
    labels3 = labels.reshape(N // BKL, 1, BKL)
    pk, pkfg3 = _tc1(lt, lt, labels3, labels3)
    hist = _sc_hist()(pk, pkfg3.reshape(M))
    out = _fin(hist)
    return out.reshape(())
